# baseline (speedup 1.0000x reference)
import functools

import jax
import jax.numpy as jnp
from jax import lax
from jax.experimental import pallas as pl
from jax.experimental.pallas import tpu as pltpu
from jax.experimental.pallas import tpu_sc as plsc

NC = 2
NS = 16
L = 16
NW = NC * NS

BATCH = 16384
EMB = 32
BPW = BATCH // NW
PH = 128
NPH = BPW // PH
HC = 64
BG = 16


def _mf_body(user_hbm, item_hbm, ue_hbm, ie_hbm, ub_hbm, ib_hbm, out_hbm,
             idx_u, idx_i, iu2a, iu2b, ii2a, ii2b, iu4, ii4,
             eu_a, eu_b, ei_a, ei_b, bu, bi, outv,
             sem_eu, sem_ei, sem_bu, sem_bi):
    wid = lax.axis_index("s") * NC + lax.axis_index("c")
    base = wid * BPW

    lane = lax.iota(jnp.int32, L)

    for ph in range(NPH):
        off = base + ph * PH
        pltpu.sync_copy(user_hbm.at[pl.ds(off, PH)], idx_u)
        pltpu.sync_copy(item_hbm.at[pl.ds(off, PH)], idx_i)

        for k in range(PH // L):
            vu = idx_u[pl.ds(k * L, L)]
            vi = idx_i[pl.ds(k * L, L)]
            iu4[pl.ds(k * L, L)] = lax.shift_right_logical(vu, 4)
            ii4[pl.ds(k * L, L)] = lax.shift_right_logical(vi, 4)
            ut = iu2a if k < 4 else iu2b
            it = ii2a if k < 4 else ii2b
            ut[pl.ds((k % 4) * L, L)] = lax.shift_right_logical(vu, 2)
            it[pl.ds((k % 4) * L, L)] = lax.shift_right_logical(vi, 2)

        c1 = pltpu.make_async_copy(ue_hbm.at[iu2a], eu_a, sem_eu)
        c2 = pltpu.make_async_copy(ue_hbm.at[iu2b], eu_b, sem_eu)
        c3 = pltpu.make_async_copy(ie_hbm.at[ii2a], ei_a, sem_ei)
        c4 = pltpu.make_async_copy(ie_hbm.at[ii2b], ei_b, sem_ei)
        c5 = pltpu.make_async_copy(ub_hbm.at[iu4], bu, sem_bu)
        c6 = pltpu.make_async_copy(ib_hbm.at[ii4], bi, sem_bi)
        c1.start(); c2.start(); c3.start(); c4.start(); c5.start(); c6.start()
        c1.wait(); c2.wait(); c3.wait(); c4.wait(); c5.wait(); c6.wait()

        def make_body(eu, ei, goff):
            def g_body(g, carry):
                rows64 = g * L + lane
                rows128 = (goff + g) * L + lane
                vu = idx_u[pl.ds((goff + g) * L, L)]
                vi = idx_i[pl.ds((goff + g) * L, L)]
                acc = plsc.load_gather(bu, [rows128, vu & (BG - 1)])
                acc = acc + plsc.load_gather(bi, [rows128, vi & (BG - 1)])
                ucb = (vu & 3) * EMB
                icb = (vi & 3) * EMB
                for d in range(EMB):
                    u = plsc.load_gather(eu, [rows64, ucb + d])
                    v = plsc.load_gather(ei, [rows64, icb + d])
                    acc = acc + u * v
                outv[pl.ds(ph * PH + (goff + g) * L, L)] = acc
                return carry
            return g_body

        lax.fori_loop(0, HC // L, make_body(eu_a, ei_a, 0), 0)
        lax.fori_loop(0, HC // L, make_body(eu_b, ei_b, HC // L), 0)

    pltpu.sync_copy(outv, out_hbm.at[pl.ds(base, BPW)])


@functools.partial(jax.jit, static_argnums=())
def _mf_call(user, item, ue4, ie4, ub16, ib16):
    mesh = plsc.VectorSubcoreMesh(core_axis_name="c", subcore_axis_name="s")
    run = pl.kernel(
        _mf_body,
        out_type=jax.ShapeDtypeStruct((BATCH,), jnp.float32),
        mesh=mesh,
        compiler_params=pltpu.CompilerParams(needs_layout_passes=False,
                                             use_tc_tiling_on_sc=False),
        scratch_types=[
            pltpu.VMEM((PH,), jnp.int32),
            pltpu.VMEM((PH,), jnp.int32),
            pltpu.VMEM((HC,), jnp.int32),
            pltpu.VMEM((HC,), jnp.int32),
            pltpu.VMEM((HC,), jnp.int32),
            pltpu.VMEM((HC,), jnp.int32),
            pltpu.VMEM((PH,), jnp.int32),
            pltpu.VMEM((PH,), jnp.int32),
            pltpu.VMEM((HC, 128), jnp.float32),
            pltpu.VMEM((HC, 128), jnp.float32),
            pltpu.VMEM((HC, 128), jnp.float32),
            pltpu.VMEM((HC, 128), jnp.float32),
            pltpu.VMEM((PH, BG), jnp.float32),
            pltpu.VMEM((PH, BG), jnp.float32),
            pltpu.VMEM((BPW,), jnp.float32),
            pltpu.SemaphoreType.DMA,
            pltpu.SemaphoreType.DMA,
            pltpu.SemaphoreType.DMA,
            pltpu.SemaphoreType.DMA,
        ],
    )
    return run(user, item, ue4, ie4, ub16, ib16)


def kernel(user, item, user_emb_w, item_emb_w, user_bias_w, item_bias_w):
    user = user.astype(jnp.int32)
    item = item.astype(jnp.int32)
    ue4 = user_emb_w.reshape(-1, 4 * EMB)
    ie4 = item_emb_w.reshape(-1, 4 * EMB)
    ub16 = user_bias_w.reshape(-1, BG)
    ib16 = item_bias_w.reshape(-1, BG)
    return _mf_call(user, item, ue4, ie4, ub16, ib16)
